# baseline (speedup 1.0000x reference)
import functools

import jax
import jax.numpy as jnp
import numpy as np
from jax import lax
from jax.experimental import pallas as pl
from jax.experimental.pallas import tpu as pltpu
from jax.experimental.pallas import tpu_sc as plsc

_FIELD_DIMS = [100000] * 26
_OFFSETS = np.concatenate(([0], np.cumsum(_FIELD_DIMS)[:-1])).astype(np.int32)

_B = 4096
_F = 26
_D = 16
_NC = 2
_NS = 16
_NW = _NC * _NS
_BPW = _B // _NW
_IPW = _BPW * _F


def _fm_body(idx_hbm, emb_hbm, lin_hbm, bias_hbm, out_hbm,
             idx_v, rows_v, lin_v, scr_v, bias_v, u_v, sem_e, sem_l):
    wid = lax.axis_index("s") * _NC + lax.axis_index("c")
    ibase = wid * _IPW
    obase = wid * _BPW

    pltpu.sync_copy(bias_hbm, bias_v)
    pltpu.sync_copy(idx_hbm.at[pl.ds(ibase, _IPW)], idx_v)
    ce = pltpu.async_copy(emb_hbm.at[idx_v], rows_v, sem_e)
    cl = pltpu.async_copy(lin_hbm.at[idx_v], lin_v, sem_l)
    ce.wait()
    cl.wait()

    bias_vec = bias_v[...]
    lane = jax.lax.iota(jnp.int32, 16)
    tail = lane >= (2 * 16 - _F)

    def group_body(g, carry):
        def row_body(j, c):
            base = (g * 16 + j) * _F
            s = rows_v[base, :]
            sq = s * s
            for f in range(1, _F):
                v = rows_v[base + f, :]
                s = s + v
                sq = sq + v * v
            t = 0.5 * (s * s - sq)
            l1 = lin_v[pl.ds(base, 16)]
            l2 = jnp.where(tail, lin_v[pl.ds(base + _F - 16, 16)], 0.0)
            u_v[j, :] = t + l1 + l2
            return c

        lax.fori_loop(0, 16, row_body, 0)
        tot = bias_vec
        for d in range(16):
            col = jnp.full((16,), d, jnp.int32)
            tot = tot + plsc.load_gather(u_v, [lane, col])
        scr_v[pl.ds(g * 16, 16)] = 1.0 / (1.0 + jnp.exp(-tot))
        return carry

    lax.fori_loop(0, _BPW // 16, group_body, 0)
    pltpu.sync_copy(scr_v, out_hbm.at[pl.ds(obase, _BPW)])


_fm_kernel = functools.partial(
    pl.kernel,
    out_type=jax.ShapeDtypeStruct((_B,), jnp.float32),
    mesh=plsc.VectorSubcoreMesh(
        core_axis_name="c", subcore_axis_name="s",
        num_cores=_NC, num_subcores=_NS),
    compiler_params=pltpu.CompilerParams(
        needs_layout_passes=False, use_tc_tiling_on_sc=False),
    scratch_types=[
        pltpu.VMEM((_IPW,), jnp.int32),
        pltpu.VMEM((_IPW, _D), jnp.float32),
        pltpu.VMEM((_IPW,), jnp.float32),
        pltpu.VMEM((_BPW,), jnp.float32),
        pltpu.VMEM((16,), jnp.float32),
        pltpu.VMEM((16, 16), jnp.float32),
        pltpu.SemaphoreType.DMA,
        pltpu.SemaphoreType.DMA,
    ],
)(_fm_body)


def kernel(x, emb_table, lin_table, lin_bias):
    idx = (x + jnp.asarray(_OFFSETS)[None, :]).reshape(-1)
    lin_flat = lin_table.reshape(-1)
    bias16 = jnp.broadcast_to(lin_bias.astype(jnp.float32), (16,))
    return _fm_kernel(idx, emb_table, lin_flat, bias16)


# speedup vs baseline: 3.6406x; 3.6406x over previous
import functools

import jax
import jax.numpy as jnp
import numpy as np
from jax import lax
from jax.experimental import pallas as pl
from jax.experimental.pallas import tpu as pltpu
from jax.experimental.pallas import tpu_sc as plsc

_FIELD_DIMS = [100000] * 26
_OFFSETS = np.concatenate(([0], np.cumsum(_FIELD_DIMS)[:-1])).astype(np.int32)

_B = 4096
_F = 26
_D = 16
_ROWS = 2600000
_NC = 2
_NS = 16
_NW = _NC * _NS
_BPW = _B // _NW
_IPW = _BPW * _F

_CW = 6144
_NSH = 423
_MAIN = _NSH * _CW
_TAIL = _ROWS - _MAIN
_NPAIR = _NW // 2
_SPS = 27
_FLD = 100000
_SLOTS = 512
_AW = _B * (_D // 2)


def _acc_body(emb_hbm, idxf_hbm, ps_hbm, pq_hbm,
              chunk_v, idx_v, hit_c, hit_b, sacc, qacc, sem_c):
    core = lax.axis_index("c")
    wid = lax.axis_index("s") * _NC + core
    band = wid & 1
    pair = wid >> 1
    lane = lax.iota(jnp.int32, 16)
    zero16 = jnp.zeros((16,), jnp.float32)

    def zbody(i, c):
        sacc[pl.ds(i * 16, 16)] = zero16
        qacc[pl.ds(i * 16, 16)] = zero16
        return c
    lax.fori_loop(0, _AW // 16, zbody, 0)

    def flush(cnt):
        ngrp = (cnt + 15) // 16

        def grp(g, c):
            mt = lane < (cnt - g * 16)
            cv = jnp.where(mt, hit_c[pl.ds(g * 16, 16)], 0)
            bv = jnp.where(mt, hit_b[pl.ds(g * 16, 16)], 0)
            b8 = bv * 8
            for dd in range(8):
                dv = jnp.full((16,), dd, jnp.int32)
                vals = jnp.where(
                    mt, plsc.load_gather(chunk_v, [dv, cv]), 0.0)
                plsc.addupdate_scatter(sacc, [b8 + dd], vals, mask=mt)
                plsc.addupdate_scatter(qacc, [b8 + dd], vals * vals, mask=mt)
            return c
        lax.fori_loop(0, ngrp, grp, 0)
        return 0

    def scan_block(blk, lo, cnt):
        def vec(i, cnt):
            iv = idx_v[pl.ds((blk * 16 + i) * 16, 16)]
            m = (iv >= lo) & (iv < lo + _CW)
            pc = plsc.all_reduce_population_count(m)[0]
            plsc.store_compressed(hit_c.at[pl.ds(cnt, 16)], iv - lo, mask=m)
            plsc.store_compressed(
                hit_b.at[pl.ds(cnt, 16)], (blk * 16 + i) * 16 + lane, mask=m)
            return cnt + pc
        return lax.fori_loop(0, 16, vec, cnt)

    def shard(k, c):
        sid = pair + k * _NPAIR

        @pl.when(sid < _NSH)
        def _():
            lo = sid * _CW
            cc = pltpu.async_copy(
                emb_hbm.at[pl.ds(band * 8, 8), pl.ds(lo, _CW)], chunk_v, sem_c)
            f0 = lo // _FLD
            f1 = (lo + _CW - 1) // _FLD
            pltpu.sync_copy(idxf_hbm.at[pl.ds(f0 * _B, _B)], idx_v)
            cc.wait()

            def blocks(blk, cnt):
                cnt = lax.cond(cnt >= 256, flush, lambda c: c, cnt)
                return scan_block(blk, lo, cnt)
            cnt = lax.fori_loop(0, 16, blocks, 0)

            def second(c):
                pltpu.sync_copy(idxf_hbm.at[pl.ds(f1 * _B, _B)], idx_v)
                return lax.fori_loop(0, 16, blocks, c)
            cnt2 = lax.cond(f1 != f0, second, lambda c: c, cnt)
            lax.cond(cnt2 > 0, flush, lambda c: 0, cnt2)
        return c
    lax.fori_loop(0, _SPS, shard, 0)

    pltpu.sync_copy(sacc, ps_hbm.at[pl.ds(wid * _AW, _AW)])
    pltpu.sync_copy(qacc, pq_hbm.at[pl.ds(wid * _AW, _AW)])


_acc_kernel = functools.partial(
    pl.kernel,
    out_type=(jax.ShapeDtypeStruct((_NW * _AW,), jnp.float32),
              jax.ShapeDtypeStruct((_NW * _AW,), jnp.float32)),
    mesh=plsc.VectorSubcoreMesh(
        core_axis_name="c", subcore_axis_name="s",
        num_cores=_NC, num_subcores=_NS),
    scratch_types=[
        pltpu.VMEM((8, _CW), jnp.float32),
        pltpu.VMEM((_B,), jnp.int32),
        pltpu.VMEM((_SLOTS + 16,), jnp.int32),
        pltpu.VMEM((_SLOTS + 16,), jnp.int32),
        pltpu.VMEM((_AW,), jnp.float32),
        pltpu.VMEM((_AW,), jnp.float32),
        pltpu.SemaphoreType.DMA,
    ],
    compiler_params=pltpu.CompilerParams(
        needs_layout_passes=False, use_tc_tiling_on_sc=True),
)(_acc_body)


def _fin_body(ps_hbm, pq_hbm, idxb_hbm, lin_hbm, tail_hbm, bias_hbm, out_hbm,
              sb0, sb1, qb0, qb1, tmp_v, idx_v, lin_v, tail_v, u_v, out_v,
              bias_v, sem_l):
    wid = lax.axis_index("s") * _NC + lax.axis_index("c")
    lane = lax.iota(jnp.int32, 16)
    b0 = wid * _BPW
    hw = _BPW * 8

    pltpu.sync_copy(bias_hbm, bias_v)
    pltpu.sync_copy(idxb_hbm.at[pl.ds(wid * _IPW, _IPW)], idx_v)
    cl = pltpu.async_copy(lin_hbm.at[idx_v], lin_v, sem_l)
    pltpu.sync_copy(tail_hbm, tail_v)

    def comb(dst, src_hbm, bnd):
        def one(w2, c):
            pltpu.sync_copy(
                src_hbm.at[pl.ds((w2 * 2 + bnd) * _AW + b0 * 8, hw)], tmp_v)

            def add(i, c2):
                dst[pl.ds(i * 16, 16)] = (
                    dst[pl.ds(i * 16, 16)] + tmp_v[pl.ds(i * 16, 16)])
                return c2
            lax.fori_loop(0, hw // 16, add, 0)
            return c
        lax.fori_loop(1, _NPAIR, one, 0)

    def init(dst, src_hbm, bnd):
        pltpu.sync_copy(src_hbm.at[pl.ds(bnd * _AW + b0 * 8, hw)], dst)

    init(sb0, ps_hbm, 0)
    init(sb1, ps_hbm, 1)
    init(qb0, pq_hbm, 0)
    init(qb1, pq_hbm, 1)
    comb(sb0, ps_hbm, 0)
    comb(sb1, ps_hbm, 1)
    comb(qb0, pq_hbm, 0)
    comb(qb1, pq_hbm, 1)

    def tscan(i, c):
        iv = idx_v[pl.ds(i * 16, 16)]
        m = iv >= _MAIN
        pc = plsc.all_reduce_population_count(m)[0]

        @pl.when(pc > 0)
        def _():
            cv = jnp.where(m, iv - _MAIN, 0)
            blv = (i * 16 + lane) // _F
            a8 = blv * 8
            for d in range(_D):
                vals = jnp.where(
                    m, plsc.load_gather(tail_v, [cv * 16 + d]), 0.0)
                sdst = sb0 if d < 8 else sb1
                qdst = qb0 if d < 8 else qb1
                plsc.addupdate_scatter(sdst, [a8 + (d % 8)], vals, mask=m)
                plsc.addupdate_scatter(qdst, [a8 + (d % 8)], vals * vals,
                                       mask=m)
        return c
    lax.fori_loop(0, _IPW // 16, tscan, 0)

    cl.wait()
    bias_vec = bias_v[...]
    tail_m = lane >= (2 * 16 - _F)
    low = lane < 8
    l8 = lane & 7

    def group(g, c):
        def row(j, c2):
            r = g * 16 + j
            a = r * 8 + l8
            sv = jnp.where(low, plsc.load_gather(sb0, [a]),
                           plsc.load_gather(sb1, [a]))
            qv = jnp.where(low, plsc.load_gather(qb0, [a]),
                           plsc.load_gather(qb1, [a]))
            u = 0.5 * (sv * sv - qv)
            base = r * _F
            l1 = lin_v[pl.ds(base, 16)]
            l2 = jnp.where(tail_m, lin_v[pl.ds(base + _F - 16, 16)], 0.0)
            u_v[j, :] = u + l1 + l2
            return c2
        lax.fori_loop(0, 16, row, 0)
        tot = bias_vec
        for d in range(16):
            col = jnp.full((16,), d, jnp.int32)
            tot = tot + plsc.load_gather(u_v, [lane, col])
        out_v[pl.ds(g * 16, 16)] = 1.0 / (1.0 + jnp.exp(-tot))
        return c
    lax.fori_loop(0, _BPW // 16, group, 0)
    pltpu.sync_copy(out_v, out_hbm.at[pl.ds(b0, _BPW)])


_fin_kernel = functools.partial(
    pl.kernel,
    out_type=jax.ShapeDtypeStruct((_B,), jnp.float32),
    mesh=plsc.VectorSubcoreMesh(
        core_axis_name="c", subcore_axis_name="s",
        num_cores=_NC, num_subcores=_NS),
    scratch_types=[
        pltpu.VMEM((_BPW * 8,), jnp.float32),
        pltpu.VMEM((_BPW * 8,), jnp.float32),
        pltpu.VMEM((_BPW * 8,), jnp.float32),
        pltpu.VMEM((_BPW * 8,), jnp.float32),
        pltpu.VMEM((_BPW * 8,), jnp.float32),
        pltpu.VMEM((_IPW,), jnp.int32),
        pltpu.VMEM((_IPW,), jnp.float32),
        pltpu.VMEM((_TAIL * _D,), jnp.float32),
        pltpu.VMEM((16, 16), jnp.float32),
        pltpu.VMEM((_BPW,), jnp.float32),
        pltpu.VMEM((16,), jnp.float32),
        pltpu.SemaphoreType.DMA,
    ],
    compiler_params=pltpu.CompilerParams(needs_layout_passes=False),
)(_fin_body)


def kernel(x, emb_table, lin_table, lin_bias):
    offs = jnp.asarray(_OFFSETS)
    idx = x + offs[None, :]
    idx_f = idx.T.reshape(-1)
    idx_b = idx.reshape(-1)
    lin_flat = lin_table.reshape(-1)
    tail = emb_table[_MAIN:, :].reshape(-1)
    bias16 = jnp.broadcast_to(lin_bias.astype(jnp.float32), (16,))
    ps, pq = _acc_kernel(emb_table.T, idx_f)
    return _fin_kernel(ps, pq, idx_b, lin_flat, tail, bias16)


# speedup vs baseline: 4.3859x; 1.2047x over previous
import functools

import jax
import jax.numpy as jnp
import numpy as np
from jax import lax
from jax.experimental import pallas as pl
from jax.experimental.pallas import tpu as pltpu
from jax.experimental.pallas import tpu_sc as plsc

_FIELD_DIMS = [100000] * 26
_OFFSETS = np.concatenate(([0], np.cumsum(_FIELD_DIMS)[:-1])).astype(np.int32)

_B = 4096
_F = 26
_D = 16
_ROWS = 2600000
_NC = 2
_NS = 16
_NW = _NC * _NS
_BPW = _B // _NW
_IPW = _BPW * _F

_CW = 3456
_NSH = 752
_MAIN = _NSH * _CW
_TAIL = _ROWS - _MAIN
_NPAIR = _NW // 2
_SPS = _NSH // _NPAIR
_FLD = 100000
_SLOTS = 384
_AW = _B * (_D // 2)
_DRN = 1024


def _acc_body(emb_hbm, idxf_hbm, ps_hbm, pq_hbm,
              chunk_a, chunk_b, idx_a, idx_b2, hit_c, hit_b, sacc, qacc,
              sem_ca, sem_cb, sem_ia, sem_ib, sem_d):
    core = lax.axis_index("c")
    wid = lax.axis_index("s") * _NC + core
    band = wid & 1
    pair = wid >> 1
    lane = lax.iota(jnp.int32, 16)
    zero16 = jnp.zeros((16,), jnp.float32)

    def zbody(i, c):
        sacc[pl.ds(i * 16, 16)] = zero16
        qacc[pl.ds(i * 16, 16)] = zero16
        return c
    lax.fori_loop(0, _AW // 16, zbody, 0)

    def start(k, chunk_v, idx_v, sem_c, sem_i):
        sid = pair + k * _NPAIR
        lo = sid * _CW
        pltpu.async_copy(
            emb_hbm.at[pl.ds(band * 8, 8), pl.ds(lo, _CW)], chunk_v, sem_c)
        pltpu.async_copy(
            idxf_hbm.at[pl.ds((lo // _FLD) * _B, _B)], idx_v, sem_i)

    def process(k, chunk_v, idx_v, sem_c, sem_i):
        sid = pair + k * _NPAIR
        lo = sid * _CW
        pltpu.make_async_copy(
            emb_hbm.at[pl.ds(band * 8, 8), pl.ds(lo, _CW)], chunk_v,
            sem_c).wait()
        pltpu.make_async_copy(
            idxf_hbm.at[pl.ds((lo // _FLD) * _B, _B)], idx_v, sem_i).wait()

        def flush(cnt):
            ngrp = (cnt + 15) // 16

            def grp(g, c):
                mt = lane < (cnt - g * 16)
                pk = hit_c[pl.ds(g * 16, 16)]
                cv = jnp.where(mt, pk & 4095, 0)
                bv = jnp.where(mt, lax.shift_right_logical(pk, 12), 0)
                b8 = bv * 8
                for dd in range(8):
                    dv = jnp.full((16,), dd, jnp.int32)
                    vals = jnp.where(
                        mt, plsc.load_gather(chunk_v, [dv, cv]), 0.0)
                    plsc.addupdate_scatter(sacc, [b8 + dd], vals, mask=mt)
                    plsc.addupdate_scatter(
                        qacc, [b8 + dd], vals * vals, mask=mt)
                return c
            lax.fori_loop(0, ngrp, grp, 0)
            return 0

        def scan_block(blk, cnt):
            def vec(i, cnt):
                p = (blk * 16 + i) * 16
                iv = idx_v[pl.ds(p, 16)]
                cvv = iv - lo
                m = cvv.astype(jnp.uint32) < jnp.uint32(_CW)
                pc = plsc.all_reduce_population_count(m)[0]
                plsc.store_compressed(
                    hit_c.at[pl.ds(cnt, 16)],
                    ((p + lane) << 12) | cvv, mask=m)
                return cnt + pc
            return lax.fori_loop(0, 16, vec, cnt)

        def blocks(blk, cnt):
            cnt = lax.cond(cnt >= 128, flush, lambda c: c, cnt)
            return scan_block(blk, cnt)
        cnt = lax.fori_loop(0, 16, blocks, 0)

        f0 = lo // _FLD
        f1 = (lo + _CW - 1) // _FLD

        def second(c):
            pltpu.sync_copy(idxf_hbm.at[pl.ds(f1 * _B, _B)], idx_v)
            return lax.fori_loop(0, 16, blocks, c)
        cnt2 = lax.cond(f1 != f0, second, lambda c: c, cnt)
        lax.cond(cnt2 > 0, flush, lambda c: 0, cnt2)

    start(0, chunk_a, idx_a, sem_ca, sem_ia)
    start(1, chunk_b, idx_b2, sem_cb, sem_ib)

    def pipe(i, c):
        k = i * 2
        process(k, chunk_a, idx_a, sem_ca, sem_ia)

        @pl.when(k + 2 < _SPS)
        def _():
            start(k + 2, chunk_a, idx_a, sem_ca, sem_ia)
        process(k + 1, chunk_b, idx_b2, sem_cb, sem_ib)

        @pl.when(k + 3 < _SPS)
        def _():
            start(k + 3, chunk_b, idx_b2, sem_cb, sem_ib)
        return c
    lax.fori_loop(0, _SPS // 2, pipe, 0)
    if _SPS % 2:
        process(_SPS - 1, chunk_a, idx_a, sem_ca, sem_ia)

    for v, (acc, dst) in enumerate(((sacc, ps_hbm), (qacc, pq_hbm))):
        def drain(w2, c):
            pltpu.async_copy(
                acc.at[pl.ds(w2 * _DRN, _DRN)],
                dst.at[pl.ds((w2 * _NW + wid) * _DRN, _DRN)], sem_d)
            return c
        lax.fori_loop(0, _NW, drain, 0)

    def wait_drain(w2, c):
        pltpu.make_async_copy(
            sacc.at[pl.ds(0, _DRN)], ps_hbm.at[pl.ds(0, _DRN)], sem_d).wait()
        pltpu.make_async_copy(
            sacc.at[pl.ds(0, _DRN)], ps_hbm.at[pl.ds(0, _DRN)], sem_d).wait()
        return c
    lax.fori_loop(0, _NW, wait_drain, 0)


_acc_kernel = functools.partial(
    pl.kernel,
    out_type=(jax.ShapeDtypeStruct((_NW * _AW,), jnp.float32),
              jax.ShapeDtypeStruct((_NW * _AW,), jnp.float32)),
    mesh=plsc.VectorSubcoreMesh(
        core_axis_name="c", subcore_axis_name="s",
        num_cores=_NC, num_subcores=_NS),
    scratch_types=[
        pltpu.VMEM((8, _CW), jnp.float32),
        pltpu.VMEM((8, _CW), jnp.float32),
        pltpu.VMEM((_B,), jnp.int32),
        pltpu.VMEM((_B,), jnp.int32),
        pltpu.VMEM((_SLOTS + 16,), jnp.int32),
        pltpu.VMEM((_SLOTS + 16,), jnp.int32),
        pltpu.VMEM((_AW,), jnp.float32),
        pltpu.VMEM((_AW,), jnp.float32),
        pltpu.SemaphoreType.DMA,
        pltpu.SemaphoreType.DMA,
        pltpu.SemaphoreType.DMA,
        pltpu.SemaphoreType.DMA,
        pltpu.SemaphoreType.DMA,
    ],
    compiler_params=pltpu.CompilerParams(
        needs_layout_passes=False, use_tc_tiling_on_sc=True),
)(_acc_body)


def _fin_body(ps_hbm, pq_hbm, idxb_hbm, lin_hbm, tail_hbm, bias_hbm, out_hbm,
              sb0, sb1, qb0, qb1, sblk, qblk, idx_v, lin_v, tail_v, u_v,
              out_v, bias_v, sem_l, sem_b):
    wid = lax.axis_index("s") * _NC + lax.axis_index("c")
    lane = lax.iota(jnp.int32, 16)
    b0 = wid * _BPW

    pltpu.sync_copy(bias_hbm, bias_v)
    pltpu.sync_copy(idxb_hbm.at[pl.ds(wid * _IPW, _IPW)], idx_v)
    cl = pltpu.async_copy(lin_hbm.at[idx_v], lin_v, sem_l)
    cs = pltpu.async_copy(
        ps_hbm.at[pl.ds(wid * _NW * _DRN, _NW * _DRN)], sblk, sem_b)
    cq = pltpu.async_copy(
        pq_hbm.at[pl.ds(wid * _NW * _DRN, _NW * _DRN)], qblk, sem_b)
    pltpu.sync_copy(tail_hbm, tail_v)
    cs.wait()
    cq.wait()

    def csum(dst, blk, bnd):
        def one(i, c):
            acc = blk[pl.ds(bnd * _DRN + i * 16, 16)]
            for j in range(1, _NPAIR):
                acc = acc + blk[pl.ds((j * 2 + bnd) * _DRN + i * 16, 16)]
            dst[pl.ds(i * 16, 16)] = acc
            return c
        lax.fori_loop(0, _DRN // 16, one, 0)

    csum(sb0, sblk, 0)
    csum(sb1, sblk, 1)
    csum(qb0, qblk, 0)
    csum(qb1, qblk, 1)

    def tscan(i, c):
        iv = idx_v[pl.ds(i * 16, 16)]
        m = iv >= _MAIN
        pc = plsc.all_reduce_population_count(m)[0]

        @pl.when(pc > 0)
        def _():
            cv = jnp.where(m, iv - _MAIN, 0)
            blv = (i * 16 + lane) // _F
            a8 = blv * 8
            for d in range(_D):
                vals = jnp.where(
                    m, plsc.load_gather(tail_v, [cv * 16 + d]), 0.0)
                sdst = sb0 if d < 8 else sb1
                qdst = qb0 if d < 8 else qb1
                plsc.addupdate_scatter(sdst, [a8 + (d % 8)], vals, mask=m)
                plsc.addupdate_scatter(qdst, [a8 + (d % 8)], vals * vals,
                                       mask=m)
        return c
    lax.fori_loop(0, _IPW // 16, tscan, 0)

    cl.wait()
    bias_vec = bias_v[...]
    tail_m = lane >= (2 * 16 - _F)
    low = lane < 8
    l8 = lane & 7

    def group(g, c):
        def row(j, c2):
            r = g * 16 + j
            a = r * 8 + l8
            sv = jnp.where(low, plsc.load_gather(sb0, [a]),
                           plsc.load_gather(sb1, [a]))
            qv = jnp.where(low, plsc.load_gather(qb0, [a]),
                           plsc.load_gather(qb1, [a]))
            u = 0.5 * (sv * sv - qv)
            base = r * _F
            l1 = lin_v[pl.ds(base, 16)]
            l2 = jnp.where(tail_m, lin_v[pl.ds(base + _F - 16, 16)], 0.0)
            u_v[j, :] = u + l1 + l2
            return c2
        lax.fori_loop(0, 16, row, 0)
        tot = bias_vec
        for d in range(16):
            col = jnp.full((16,), d, jnp.int32)
            tot = tot + plsc.load_gather(u_v, [lane, col])
        out_v[pl.ds(g * 16, 16)] = 1.0 / (1.0 + jnp.exp(-tot))
        return c
    lax.fori_loop(0, _BPW // 16, group, 0)
    pltpu.sync_copy(out_v, out_hbm.at[pl.ds(b0, _BPW)])


_fin_kernel = functools.partial(
    pl.kernel,
    out_type=jax.ShapeDtypeStruct((_B,), jnp.float32),
    mesh=plsc.VectorSubcoreMesh(
        core_axis_name="c", subcore_axis_name="s",
        num_cores=_NC, num_subcores=_NS),
    scratch_types=[
        pltpu.VMEM((_BPW * 8,), jnp.float32),
        pltpu.VMEM((_BPW * 8,), jnp.float32),
        pltpu.VMEM((_BPW * 8,), jnp.float32),
        pltpu.VMEM((_BPW * 8,), jnp.float32),
        pltpu.VMEM((_NW * _DRN,), jnp.float32),
        pltpu.VMEM((_NW * _DRN,), jnp.float32),
        pltpu.VMEM((_IPW,), jnp.int32),
        pltpu.VMEM((_IPW,), jnp.float32),
        pltpu.VMEM((_TAIL * _D,), jnp.float32),
        pltpu.VMEM((16, 16), jnp.float32),
        pltpu.VMEM((_BPW,), jnp.float32),
        pltpu.VMEM((16,), jnp.float32),
        pltpu.SemaphoreType.DMA,
        pltpu.SemaphoreType.DMA,
    ],
    compiler_params=pltpu.CompilerParams(needs_layout_passes=False),
)(_fin_body)


def kernel(x, emb_table, lin_table, lin_bias):
    offs = jnp.asarray(_OFFSETS)
    idx = x + offs[None, :]
    idx_f = idx.T.reshape(-1)
    idx_b = idx.reshape(-1)
    lin_flat = lin_table.reshape(-1)
    tail = emb_table[_MAIN:, :].reshape(-1)
    bias16 = jnp.broadcast_to(lin_bias.astype(jnp.float32), (16,))
    ps, pq = _acc_kernel(emb_table.T, idx_f)
    return _fin_kernel(ps, pq, idx_b, lin_flat, tail, bias16)


# speedup vs baseline: 4.6330x; 1.0564x over previous
import functools

import jax
import jax.numpy as jnp
import numpy as np
from jax import lax
from jax.experimental import pallas as pl
from jax.experimental.pallas import tpu as pltpu
from jax.experimental.pallas import tpu_sc as plsc

_FIELD_DIMS = [100000] * 26
_OFFSETS = np.concatenate(([0], np.cumsum(_FIELD_DIMS)[:-1])).astype(np.int32)

_B = 4096
_F = 26
_D = 16
_ROWS = 2600000
_NC = 2
_NS = 16
_NW = _NC * _NS
_BPW = _B // _NW
_IPW = _BPW * _F

_CW = 3456
_NSH = 752
_MAIN = _NSH * _CW
_TAIL = _ROWS - _MAIN
_NPAIR = _NW // 2
_SPS = _NSH // _NPAIR
_FLD = 100000
_SLOTS = 384
_AW = _B * (_D // 2)
_DRN = 1024


def _acc_body(emb_hbm, idxf_hbm, ps_hbm, pq_hbm,
              chunk_a, chunk_b, idx_a, idx_b2, hit_c, hit_b, sacc, qacc,
              sem_ca, sem_cb, sem_ia, sem_ib, sem_d):
    core = lax.axis_index("c")
    wid = lax.axis_index("s") * _NC + core
    band = wid & 1
    pair = wid >> 1
    lane = lax.iota(jnp.int32, 16)
    zero16 = jnp.zeros((16,), jnp.float32)

    def zbody(i, c):
        sacc[pl.ds(i * 16, 16)] = zero16
        qacc[pl.ds(i * 16, 16)] = zero16
        return c
    lax.fori_loop(0, _AW // 16, zbody, 0)

    def start(k, chunk_v, idx_v, sem_c, sem_i):
        sid = pair + k * _NPAIR
        lo = sid * _CW
        pltpu.async_copy(
            emb_hbm.at[pl.ds(band * 8, 8), pl.ds(lo, _CW)], chunk_v, sem_c)
        pltpu.async_copy(
            idxf_hbm.at[pl.ds((lo // _FLD) * _B, _B)], idx_v, sem_i)

    def process(k, chunk_v, idx_v, sem_c, sem_i):
        sid = pair + k * _NPAIR
        lo = sid * _CW
        pltpu.make_async_copy(
            emb_hbm.at[pl.ds(band * 8, 8), pl.ds(lo, _CW)], chunk_v,
            sem_c).wait()
        pltpu.make_async_copy(
            idxf_hbm.at[pl.ds((lo // _FLD) * _B, _B)], idx_v, sem_i).wait()

        def flush(cnt):
            ngrp = (cnt + 15) // 16

            def grp(g, c):
                mt = lane < (cnt - g * 16)
                pk = hit_c[pl.ds(g * 16, 16)]
                cv = jnp.where(mt, pk & 4095, 0)
                bv = jnp.where(mt, lax.shift_right_logical(pk, 12), 0)
                b8 = bv * 8
                for dd in range(8):
                    dv = jnp.full((16,), dd, jnp.int32)
                    vals = jnp.where(
                        mt, plsc.load_gather(chunk_v, [dv, cv]), 0.0)
                    plsc.addupdate_scatter(sacc, [b8 + dd], vals, mask=mt)
                    plsc.addupdate_scatter(
                        qacc, [b8 + dd], vals * vals, mask=mt)
                return c
            lax.fori_loop(0, ngrp, grp, 0)
            return 0

        def scan_block(blk, cnt):
            def quad(i, cnt):
                for q in range(4):
                    p = (blk * 16 + i * 4 + q) * 16
                    iv = idx_v[pl.ds(p, 16)]
                    cvv = iv - lo
                    m = cvv.astype(jnp.uint32) < jnp.uint32(_CW)
                    pc = plsc.all_reduce_population_count(m)[0]
                    plsc.store_compressed(
                        hit_c.at[pl.ds(cnt, 16)],
                        ((p + lane) << 12) | cvv, mask=m)
                    cnt = cnt + pc
                return cnt
            return lax.fori_loop(0, 4, quad, cnt)

        def blocks(blk, cnt):
            cnt = lax.cond(cnt >= 128, flush, lambda c: c, cnt)
            return scan_block(blk, cnt)
        cnt = lax.fori_loop(0, 16, blocks, 0)

        f0 = lo // _FLD
        f1 = (lo + _CW - 1) // _FLD

        def second(c):
            pltpu.sync_copy(idxf_hbm.at[pl.ds(f1 * _B, _B)], idx_v)
            return lax.fori_loop(0, 16, blocks, c)
        cnt2 = lax.cond(f1 != f0, second, lambda c: c, cnt)
        lax.cond(cnt2 > 0, flush, lambda c: 0, cnt2)

    start(0, chunk_a, idx_a, sem_ca, sem_ia)
    start(1, chunk_b, idx_b2, sem_cb, sem_ib)

    def pipe(i, c):
        k = i * 2
        process(k, chunk_a, idx_a, sem_ca, sem_ia)

        @pl.when(k + 2 < _SPS)
        def _():
            start(k + 2, chunk_a, idx_a, sem_ca, sem_ia)
        process(k + 1, chunk_b, idx_b2, sem_cb, sem_ib)

        @pl.when(k + 3 < _SPS)
        def _():
            start(k + 3, chunk_b, idx_b2, sem_cb, sem_ib)
        return c
    lax.fori_loop(0, _SPS // 2, pipe, 0)
    if _SPS % 2:
        process(_SPS - 1, chunk_a, idx_a, sem_ca, sem_ia)

    for v, (acc, dst) in enumerate(((sacc, ps_hbm), (qacc, pq_hbm))):
        def drain(w2, c):
            pltpu.async_copy(
                acc.at[pl.ds(w2 * _DRN, _DRN)],
                dst.at[pl.ds((w2 * _NW + wid) * _DRN, _DRN)], sem_d)
            return c
        lax.fori_loop(0, _NW, drain, 0)

    def wait_drain(w2, c):
        pltpu.make_async_copy(
            sacc.at[pl.ds(0, _DRN)], ps_hbm.at[pl.ds(0, _DRN)], sem_d).wait()
        pltpu.make_async_copy(
            sacc.at[pl.ds(0, _DRN)], ps_hbm.at[pl.ds(0, _DRN)], sem_d).wait()
        return c
    lax.fori_loop(0, _NW, wait_drain, 0)


_acc_kernel = functools.partial(
    pl.kernel,
    out_type=(jax.ShapeDtypeStruct((_NW * _AW,), jnp.float32),
              jax.ShapeDtypeStruct((_NW * _AW,), jnp.float32)),
    mesh=plsc.VectorSubcoreMesh(
        core_axis_name="c", subcore_axis_name="s",
        num_cores=_NC, num_subcores=_NS),
    scratch_types=[
        pltpu.VMEM((8, _CW), jnp.float32),
        pltpu.VMEM((8, _CW), jnp.float32),
        pltpu.VMEM((_B,), jnp.int32),
        pltpu.VMEM((_B,), jnp.int32),
        pltpu.VMEM((_SLOTS + 16,), jnp.int32),
        pltpu.VMEM((_SLOTS + 16,), jnp.int32),
        pltpu.VMEM((_AW,), jnp.float32),
        pltpu.VMEM((_AW,), jnp.float32),
        pltpu.SemaphoreType.DMA,
        pltpu.SemaphoreType.DMA,
        pltpu.SemaphoreType.DMA,
        pltpu.SemaphoreType.DMA,
        pltpu.SemaphoreType.DMA,
    ],
    compiler_params=pltpu.CompilerParams(
        needs_layout_passes=False, use_tc_tiling_on_sc=True),
)(_acc_body)


def _fin_body(ps_hbm, pq_hbm, idxb_hbm, lin_hbm, tail_hbm, bias_hbm, out_hbm,
              sb0, sb1, qb0, qb1, sblk, qblk, idx_v, lin_v, tail_v, u_v,
              out_v, bias_v, sem_l, sem_b):
    wid = lax.axis_index("s") * _NC + lax.axis_index("c")
    lane = lax.iota(jnp.int32, 16)
    b0 = wid * _BPW

    pltpu.sync_copy(bias_hbm, bias_v)
    pltpu.sync_copy(idxb_hbm.at[pl.ds(wid * _IPW, _IPW)], idx_v)
    cl = pltpu.async_copy(lin_hbm.at[idx_v], lin_v, sem_l)
    cs = pltpu.async_copy(
        ps_hbm.at[pl.ds(wid * _NW * _DRN, _NW * _DRN)], sblk, sem_b)
    cq = pltpu.async_copy(
        pq_hbm.at[pl.ds(wid * _NW * _DRN, _NW * _DRN)], qblk, sem_b)
    pltpu.sync_copy(tail_hbm, tail_v)
    cs.wait()
    cq.wait()

    def csum(dst, blk, bnd):
        def one(i, c):
            acc = blk[pl.ds(bnd * _DRN + i * 16, 16)]
            for j in range(1, _NPAIR):
                acc = acc + blk[pl.ds((j * 2 + bnd) * _DRN + i * 16, 16)]
            dst[pl.ds(i * 16, 16)] = acc
            return c
        lax.fori_loop(0, _DRN // 16, one, 0)

    csum(sb0, sblk, 0)
    csum(sb1, sblk, 1)
    csum(qb0, qblk, 0)
    csum(qb1, qblk, 1)

    def tscan(i, c):
        iv = idx_v[pl.ds(i * 16, 16)]
        m = iv >= _MAIN
        pc = plsc.all_reduce_population_count(m)[0]

        @pl.when(pc > 0)
        def _():
            cv = jnp.where(m, iv - _MAIN, 0)
            blv = (i * 16 + lane) // _F
            a8 = blv * 8
            for d in range(_D):
                vals = jnp.where(
                    m, plsc.load_gather(tail_v, [cv * 16 + d]), 0.0)
                sdst = sb0 if d < 8 else sb1
                qdst = qb0 if d < 8 else qb1
                plsc.addupdate_scatter(sdst, [a8 + (d % 8)], vals, mask=m)
                plsc.addupdate_scatter(qdst, [a8 + (d % 8)], vals * vals,
                                       mask=m)
        return c
    lax.fori_loop(0, _IPW // 16, tscan, 0)

    cl.wait()
    bias_vec = bias_v[...]
    tail_m = lane >= (2 * 16 - _F)
    low = lane < 8
    l8 = lane & 7

    def group(g, c):
        def row(j, c2):
            r = g * 16 + j
            a = r * 8 + l8
            sv = jnp.where(low, plsc.load_gather(sb0, [a]),
                           plsc.load_gather(sb1, [a]))
            qv = jnp.where(low, plsc.load_gather(qb0, [a]),
                           plsc.load_gather(qb1, [a]))
            u = 0.5 * (sv * sv - qv)
            base = r * _F
            l1 = lin_v[pl.ds(base, 16)]
            l2 = jnp.where(tail_m, lin_v[pl.ds(base + _F - 16, 16)], 0.0)
            u_v[j, :] = u + l1 + l2
            return c2
        lax.fori_loop(0, 16, row, 0)
        tot = bias_vec
        for d in range(16):
            col = jnp.full((16,), d, jnp.int32)
            tot = tot + plsc.load_gather(u_v, [lane, col])
        out_v[pl.ds(g * 16, 16)] = 1.0 / (1.0 + jnp.exp(-tot))
        return c
    lax.fori_loop(0, _BPW // 16, group, 0)
    pltpu.sync_copy(out_v, out_hbm.at[pl.ds(b0, _BPW)])


_fin_kernel = functools.partial(
    pl.kernel,
    out_type=jax.ShapeDtypeStruct((_B,), jnp.float32),
    mesh=plsc.VectorSubcoreMesh(
        core_axis_name="c", subcore_axis_name="s",
        num_cores=_NC, num_subcores=_NS),
    scratch_types=[
        pltpu.VMEM((_BPW * 8,), jnp.float32),
        pltpu.VMEM((_BPW * 8,), jnp.float32),
        pltpu.VMEM((_BPW * 8,), jnp.float32),
        pltpu.VMEM((_BPW * 8,), jnp.float32),
        pltpu.VMEM((_NW * _DRN,), jnp.float32),
        pltpu.VMEM((_NW * _DRN,), jnp.float32),
        pltpu.VMEM((_IPW,), jnp.int32),
        pltpu.VMEM((_IPW,), jnp.float32),
        pltpu.VMEM((_TAIL * _D,), jnp.float32),
        pltpu.VMEM((16, 16), jnp.float32),
        pltpu.VMEM((_BPW,), jnp.float32),
        pltpu.VMEM((16,), jnp.float32),
        pltpu.SemaphoreType.DMA,
        pltpu.SemaphoreType.DMA,
    ],
    compiler_params=pltpu.CompilerParams(needs_layout_passes=False),
)(_fin_body)


def kernel(x, emb_table, lin_table, lin_bias):
    offs = jnp.asarray(_OFFSETS)
    idx = x + offs[None, :]
    idx_f = idx.T.reshape(-1)
    idx_b = idx.reshape(-1)
    lin_flat = lin_table.reshape(-1)
    tail = emb_table[_MAIN:, :].reshape(-1)
    bias16 = jnp.broadcast_to(lin_bias.astype(jnp.float32), (16,))
    ps, pq = _acc_kernel(emb_table.T, idx_f)
    return _fin_kernel(ps, pq, idx_b, lin_flat, tail, bias16)
